# Initial kernel scaffold; baseline (speedup 1.0000x reference)
import jax
import jax.numpy as jnp
from jax.experimental import pallas as pl


def kernel(x, edge_index, bn1_w, bn1_b, conv1_W, conv1_b, bn2_w, bn2_b, conv2_W, conv2_b, bn3_w, bn3_b, fc1_W, fc1_b, fc2_W, fc2_b):
    raise NotImplementedError("write your pallas kernel here")


# speedup vs baseline: 1.5039x; 1.5039x over previous
import functools

import jax
import jax.numpy as jnp
from jax import lax
from jax.experimental import pallas as pl
from jax.experimental.pallas import tpu as pltpu
from jax.experimental.pallas import tpu_sc as plsc

N = 10000
E = 160000
EPS = 1e-5
DROP_PROB = 0.2

NT = 10240
E1 = E + N
EPAD = 172032
NBLK = 84
NHALF = 42
KE = 128
NBLK_A = 42
TILE_E = EPAD // 16
TILE_EA = EPAD // 32
ZROW = N
NSUB = 16
NCORE = 2


def _sc_mesh():
    return plsc.VectorSubcoreMesh(
        core_axis_name="c", subcore_axis_name="s",
        num_cores=NCORE, num_subcores=NSUB)


def _sc_prep_body(row_hbm, ew_hbm, cola_hbm, zflat_hbm,
                  degp_hbm, rowp4_hbm,
                  deg_sh, row_v, ew_v, col_v, base_v, dbuf):
    cid = lax.axis_index("c")
    sid = lax.axis_index("s")
    w = cid * NSUB + sid
    off = w * TILE_EA
    pltpu.sync_copy(row_hbm.at[pl.ds(off, TILE_EA)], row_v)
    pltpu.sync_copy(ew_hbm.at[pl.ds(off, TILE_EA)], ew_v)
    pltpu.sync_copy(cola_hbm.at[w], col_v)
    pltpu.sync_copy(zflat_hbm.at[pl.ds(sid * (NT // NSUB), NT // NSUB)],
                    deg_sh.at[pl.ds(sid * (NT // NSUB), NT // NSUB)])
    plsc.subcore_barrier()

    def scat(j, _):
        pltpu.sync_copy(ew_v.at[pl.ds(j * 128, 128)],
                        deg_sh.at[col_v.at[j]], add=True)
        return 0
    lax.fori_loop(0, NBLK_A, scat, 0)

    def remap(i, _):
        sl = pl.ds(i * 16, 16)
        m = ew_v[sl] > 0.0
        base_v[sl] = jnp.where(m, row_v[sl], jnp.full((16,), ZROW, jnp.int32))
        return 0
    lax.fori_loop(0, TILE_EA // 16, remap, 0)
    for v in range(4):
        pltpu.sync_copy(base_v, rowp4_hbm.at[pl.ds(v * EPAD + off, TILE_EA)])
        if v < 3:
            def bump(i, _):
                sl = pl.ds(i * 16, 16)
                base_v[sl] = base_v[sl] + NT
                return 0
            lax.fori_loop(0, TILE_EA // 16, bump, 0)

    plsc.subcore_barrier()
    sl = pl.ds(sid * (NT // NSUB), NT // NSUB)
    pltpu.sync_copy(deg_sh.at[sl], dbuf)
    pltpu.sync_copy(dbuf, degp_hbm.at[pl.ds(cid * NT + sid * (NT // NSUB),
                                            NT // NSUB)])


def _sc_prep(rowf, ewf, col_a, zflat):
    return pl.kernel(
        _sc_prep_body,
        out_type=[jax.ShapeDtypeStruct((NCORE * NT,), jnp.float32),
                  jax.ShapeDtypeStruct((4 * EPAD,), jnp.int32)],
        mesh=_sc_mesh(),
        scratch_types=[
            pltpu.MemorySpace.VMEM_SHARED((NT,), jnp.float32),
            pltpu.MemorySpace.VMEM((TILE_EA,), jnp.int32),
            pltpu.MemorySpace.VMEM((TILE_EA,), jnp.float32),
            pltpu.MemorySpace.VMEM((NBLK_A, 128), jnp.int32),
            pltpu.MemorySpace.VMEM((TILE_EA,), jnp.int32),
            pltpu.MemorySpace.VMEM((NT // NSUB,), jnp.float32),
        ],
    )(rowf, ewf, col_a, zflat)


def _sc_agg_body(h_hbm, rowp_hbm, col_hbm, zrows_hbm,
                 agg_hbm,
                 acc_sh, row_v, col_v, rbuf, sem0, sem1):
    cid = lax.axis_index("c")
    sid = lax.axis_index("s")
    rows_per_tile = NT // NSUB
    sems = (sem0, sem1)
    for cc in range(2):
        c = 2 * cid + cc
        for k in range(rows_per_tile // 128):
            pltpu.sync_copy(zrows_hbm,
                            acc_sh.at[pl.ds(sid * rows_per_tile + k * 128, 128)])
        plsc.subcore_barrier()

        def gather_start(j, b):
            pltpu.async_copy(h_hbm.at[row_v.at[j]], rbuf.at[b], sems[b])

        def gather_wait(j, b):
            pltpu.make_async_copy(h_hbm.at[row_v.at[j]], rbuf.at[b],
                                  sems[b]).wait()

        def scat(j, b):
            pltpu.sync_copy(rbuf.at[b], acc_sh.at[col_v.at[j]], add=True)

        for hh in range(2):
            pltpu.sync_copy(rowp_hbm.at[c, sid, hh], row_v)
            pltpu.sync_copy(col_hbm.at[sid, hh], col_v)
            gather_start(0, 0)

            def step(jj, _):
                j0 = 2 * jj
                gather_start(j0 + 1, 1)
                gather_wait(j0, 0)
                scat(j0, 0)

                @pl.when(jj < NHALF // 2 - 1)
                def _():
                    gather_start(j0 + 2, 0)
                gather_wait(j0 + 1, 1)
                scat(j0 + 1, 1)
                return 0
            lax.fori_loop(0, NHALF // 2, step, 0)
        plsc.subcore_barrier()
        for k in range(rows_per_tile // KE):
            sl = pl.ds(sid * rows_per_tile + k * KE, KE)
            pltpu.sync_copy(acc_sh.at[sl], rbuf.at[0])
            pltpu.sync_copy(rbuf.at[0], agg_hbm.at[c, sl])
        plsc.subcore_barrier()


def _sc_agg(hflat, rowp4r, col3, zrows):
    return pl.kernel(
        _sc_agg_body,
        out_type=jax.ShapeDtypeStruct((4, NT, 128), jnp.float32),
        mesh=_sc_mesh(),
        scratch_types=[
            pltpu.MemorySpace.VMEM_SHARED((NT, 128), jnp.float32),
            pltpu.MemorySpace.VMEM((NHALF, 128), jnp.int32),
            pltpu.MemorySpace.VMEM((NHALF, 128), jnp.int32),
            pltpu.MemorySpace.VMEM((2, KE, 128), jnp.float32),
            pltpu.SemaphoreType.DMA,
            pltpu.SemaphoreType.DMA,
        ],
    )(hflat, rowp4r, col3, zrows)


def _tc_prep_body(x_ref, degp_ref, s1_ref, s2_ref, dinv_ref):
    x = x_ref[...]
    s1_ref[...] = jnp.sum(x, axis=0, keepdims=True)
    s2_ref[...] = jnp.sum(x * x, axis=0, keepdims=True)
    deg = degp_ref[0] + degp_ref[1]
    rid = (lax.broadcasted_iota(jnp.int32, (NT // 128, 128), 0) * 128
           + lax.broadcasted_iota(jnp.int32, (NT // 128, 128), 1))
    valid = (rid < N) & (deg > 0.0)
    dinv_ref[...] = jnp.where(valid, lax.rsqrt(jnp.maximum(deg, 1e-12)), 0.0)


def _tc_prep(xp, degp2):
    return pl.pallas_call(
        _tc_prep_body,
        out_shape=[jax.ShapeDtypeStruct((1, 128), jnp.float32),
                   jax.ShapeDtypeStruct((1, 128), jnp.float32),
                   jax.ShapeDtypeStruct((NT // 128, 128), jnp.float32)],
    )(xp, degp2)


def _bn_affine(s1, s2, w, b):
    mean = s1 / N
    var = s2 / N - mean * mean
    alpha = w * lax.rsqrt(var + EPS)
    beta = b - mean * alpha
    return alpha, beta


def _tc_mm1_body(x_ref, s1_ref, s2_ref, w_ref, b_ref, w1_ref, dinv_ref,
                 out_ref):
    alpha, beta = _bn_affine(s1_ref[...], s2_ref[...], w_ref[...], b_ref[...])
    xn = x_ref[...] * alpha + beta
    h = jnp.dot(xn, w1_ref[...], preferred_element_type=jnp.float32)
    h = h * dinv_ref[...]
    for c in range(4):
        out_ref[c] = h[:, c * 128:(c + 1) * 128]


def _tc_mm1(xp, s1x, s2x, bn1wp, bn1bp, w1p, dinv_c):
    g = NT // 128
    return pl.pallas_call(
        _tc_mm1_body,
        grid=(g,),
        in_specs=[
            pl.BlockSpec((128, 128), lambda i: (i, 0)),
            pl.BlockSpec((1, 128), lambda i: (0, 0)),
            pl.BlockSpec((1, 128), lambda i: (0, 0)),
            pl.BlockSpec((1, 128), lambda i: (0, 0)),
            pl.BlockSpec((1, 128), lambda i: (0, 0)),
            pl.BlockSpec((128, 512), lambda i: (0, 0)),
            pl.BlockSpec((128, 1), lambda i: (i, 0)),
        ],
        out_specs=pl.BlockSpec((4, 128, 128), lambda i: (0, i, 0)),
        out_shape=jax.ShapeDtypeStruct((4, NT, 128), jnp.float32),
    )(xp, s1x, s2x, bn1wp, bn1bp, w1p, dinv_c)


def _assemble(agg):
    return jnp.concatenate([agg[c] for c in range(4)], axis=1)


def _tc_stats1_body(agg_ref, dinv_ref, b_ref, s1_ref, s2_ref):
    i = pl.program_id(0)
    t = jnp.maximum(_assemble(agg_ref[...]) * dinv_ref[...] + b_ref[...], 0.0)
    rid = i * 128 + lax.broadcasted_iota(jnp.int32, (128, 1), 0)
    t = jnp.where(rid < N, t, 0.0)

    @pl.when(i == 0)
    def _():
        s1_ref[...] = jnp.zeros_like(s1_ref)
        s2_ref[...] = jnp.zeros_like(s2_ref)
    s1_ref[...] += jnp.sum(t, axis=0, keepdims=True)
    s2_ref[...] += jnp.sum(t * t, axis=0, keepdims=True)


def _tc_stats1(agg, dinv_c, br):
    g = NT // 128
    return pl.pallas_call(
        _tc_stats1_body,
        grid=(g,),
        in_specs=[
            pl.BlockSpec((4, 128, 128), lambda i: (0, i, 0)),
            pl.BlockSpec((128, 1), lambda i: (i, 0)),
            pl.BlockSpec((1, 512), lambda i: (0, 0)),
        ],
        out_specs=[pl.BlockSpec((1, 512), lambda i: (0, 0)),
                   pl.BlockSpec((1, 512), lambda i: (0, 0))],
        out_shape=[jax.ShapeDtypeStruct((1, 512), jnp.float32),
                   jax.ShapeDtypeStruct((1, 512), jnp.float32)],
    )(agg, dinv_c, br)


def _tc_mm2_body(agg_ref, dinv_ref, b1_ref, s1_ref, s2_ref, w_ref, b_ref,
                 w2_ref, out_ref):
    dinv = dinv_ref[...]
    t = jnp.maximum(_assemble(agg_ref[...]) * dinv + b1_ref[...], 0.0)
    alpha, beta = _bn_affine(s1_ref[...], s2_ref[...], w_ref[...], b_ref[...])
    tn = t * alpha + beta
    h = jnp.dot(tn, w2_ref[...], preferred_element_type=jnp.float32)
    h = h * dinv
    for c in range(4):
        out_ref[c] = h[:, c * 128:(c + 1) * 128]


def _tc_mm2(agg, dinv_c, b1r, s1, s2, bn2wr, bn2br, w2):
    g = NT // 128
    bs_full = lambda shape: pl.BlockSpec(shape, lambda i: tuple(0 for _ in shape))
    return pl.pallas_call(
        _tc_mm2_body,
        grid=(g,),
        in_specs=[
            pl.BlockSpec((4, 128, 128), lambda i: (0, i, 0)),
            pl.BlockSpec((128, 1), lambda i: (i, 0)),
            bs_full((1, 512)), bs_full((1, 512)), bs_full((1, 512)),
            bs_full((1, 512)), bs_full((1, 512)),
            bs_full((512, 512)),
        ],
        out_specs=pl.BlockSpec((4, 128, 128), lambda i: (0, i, 0)),
        out_shape=jax.ShapeDtypeStruct((4, NT, 128), jnp.float32),
    )(agg, dinv_c, b1r, s1, s2, bn2wr, bn2br, w2)


def _tc_stats2_body(agg_ref, dinv_ref, b_ref, s1_ref, s2_ref, mx_ref, mn_ref):
    i = pl.program_id(0)
    u = jnp.maximum(_assemble(agg_ref[...]) * dinv_ref[...] + b_ref[...], 0.0)
    rid = i * 128 + lax.broadcasted_iota(jnp.int32, (128, 1), 0)
    valid = rid < N
    uz = jnp.where(valid, u, 0.0)
    ux = jnp.where(valid, u, -3e38)
    un = jnp.where(valid, u, 3e38)

    @pl.when(i == 0)
    def _():
        s1_ref[...] = jnp.zeros_like(s1_ref)
        s2_ref[...] = jnp.zeros_like(s2_ref)
        mx_ref[...] = jnp.full_like(mx_ref, -3e38)
        mn_ref[...] = jnp.full_like(mn_ref, 3e38)
    s1_ref[...] += jnp.sum(uz, axis=0, keepdims=True)
    s2_ref[...] += jnp.sum(uz * uz, axis=0, keepdims=True)
    mx_ref[...] = jnp.maximum(mx_ref[...], jnp.max(ux, axis=0, keepdims=True))
    mn_ref[...] = jnp.minimum(mn_ref[...], jnp.min(un, axis=0, keepdims=True))


def _tc_stats2(agg, dinv_c, br):
    g = NT // 128
    return pl.pallas_call(
        _tc_stats2_body,
        grid=(g,),
        in_specs=[
            pl.BlockSpec((4, 128, 128), lambda i: (0, i, 0)),
            pl.BlockSpec((128, 1), lambda i: (i, 0)),
            pl.BlockSpec((1, 512), lambda i: (0, 0)),
        ],
        out_specs=[pl.BlockSpec((1, 512), lambda i: (0, 0))] * 4,
        out_shape=[jax.ShapeDtypeStruct((1, 512), jnp.float32)] * 4,
    )(agg, dinv_c, br)


def _tc_final_body(s1_ref, s2_ref, mx_ref, mn_ref, w3_ref, b3_ref,
                   fc1w_ref, fc1b_ref, fc2w_ref, fc2b_ref, out_ref):
    alpha, beta = _bn_affine(s1_ref[...], s2_ref[...], w3_ref[...], b3_ref[...])
    g0 = jnp.where(alpha >= 0.0, alpha * mx_ref[...], alpha * mn_ref[...]) + beta
    g1 = jnp.dot(g0, fc1w_ref[...], preferred_element_type=jnp.float32)
    g1 = jnp.maximum(g1 + fc1b_ref[...], 0.0)
    g2 = jnp.sum(g1 * fc2w_ref[...], axis=1, keepdims=True) + fc2b_ref[...]
    out_ref[...] = jax.nn.sigmoid(g2)


def _tc_final(s1, s2, mx, mn, bn3wr, bn3br, fc1w, fc1br, fc2wr, fc2br):
    return pl.pallas_call(
        _tc_final_body,
        out_shape=jax.ShapeDtypeStruct((1, 1), jnp.float32),
    )(s1, s2, mx, mn, bn3wr, bn3br, fc1w, fc1br, fc2wr, fc2br)


def kernel(x, edge_index, bn1_w, bn1_b, conv1_W, conv1_b, bn2_w, bn2_b,
           conv2_W, conv2_b, bn3_w, bn3_b, fc1_W, fc1_b, fc2_W, fc2_b):
    f32 = jnp.float32
    i32 = jnp.int32
    mask = jax.random.uniform(jax.random.key(42), (E,)) > DROP_PROB
    loop = jnp.arange(N, dtype=i32)
    row = jnp.concatenate([edge_index[0].astype(i32), loop])
    col = jnp.concatenate([edge_index[1].astype(i32), loop])
    ew = jnp.concatenate([mask.astype(f32), jnp.ones((N,), f32)])
    pad = EPAD - E1
    rowf = jnp.pad(row, (0, pad))
    colf = jnp.pad(col, (0, pad), constant_values=ZROW)
    ewf = jnp.pad(ew, (0, pad))
    col_a = colf.reshape(32, NBLK_A, 128)
    col3 = colf.reshape(16, 2, NHALF, 128)
    zflat = jnp.zeros((NT,), f32)
    zrows = jnp.zeros((128, 128), f32)

    degp, rowp4 = _sc_prep(rowf, ewf, col_a, zflat)
    rowp4r = rowp4.reshape(4, 16, 2, NHALF, 128)

    xp = jnp.pad(x, ((0, NT - N), (0, 128 - x.shape[1])))
    w1p = jnp.pad(conv1_W, ((0, 128 - conv1_W.shape[0]), (0, 0)))
    bn1wp = jnp.pad(bn1_w, (0, 128 - bn1_w.shape[0])).reshape(1, 128)
    bn1bp = jnp.pad(bn1_b, (0, 128 - bn1_b.shape[0])).reshape(1, 128)
    s1x, s2x, dinv2d = _tc_prep(xp, degp.reshape(2, NT // 128, 128))
    dinv_c = dinv2d.reshape(NT, 1)

    b1r = conv1_b.reshape(1, 512)
    b2r = conv2_b.reshape(1, 512)
    bn2wr = bn2_w.reshape(1, 512)
    bn2br = bn2_b.reshape(1, 512)
    bn3wr = bn3_w.reshape(1, 512)
    bn3br = bn3_b.reshape(1, 512)
    fc1br = fc1_b.reshape(1, 512)
    fc2wr = fc2_W.reshape(1, 512)
    fc2br = fc2_b.reshape(1, 1)

    h1 = _tc_mm1(xp, s1x, s2x, bn1wp, bn1bp, w1p, dinv_c)
    agg1 = _sc_agg(h1.reshape(4 * NT, 128), rowp4r, col3, zrows)
    s1, s2 = _tc_stats1(agg1, dinv_c, b1r)
    h2 = _tc_mm2(agg1, dinv_c, b1r, s1, s2, bn2wr, bn2br, conv2_W)
    agg2 = _sc_agg(h2.reshape(4 * NT, 128), rowp4r, col3, zrows)
    s1b, s2b, mx, mn = _tc_stats2(agg2, dinv_c, b2r)
    return _tc_final(s1b, s2b, mx, mn, bn3wr, bn3br, fc1_W, fc1br,
                     fc2wr, fc2br)


# speedup vs baseline: 3.5336x; 2.3496x over previous
import functools

import jax
import jax.numpy as jnp
from jax import lax
from jax.experimental import pallas as pl
from jax.experimental.pallas import tpu as pltpu
from jax.experimental.pallas import tpu_sc as plsc

N = 10000
E = 160000
EPS = 1e-5
DROP_PROB = 0.2

NT = 10240
EKEEP = 127863
E1 = EKEEP + N
EPAD = 139264
NBLK = 68
NHALF = 34
KE = 128
NBLK_A = 34
TILE_E = EPAD // 16
TILE_EA = EPAD // 32
ZROW = N
NSUB = 16
NCORE = 2


def _sc_mesh():
    return plsc.VectorSubcoreMesh(
        core_axis_name="c", subcore_axis_name="s",
        num_cores=NCORE, num_subcores=NSUB)


def _sc_prep_body(row_hbm, ew_hbm, cola_hbm, zflat_hbm,
                  degp_hbm, rowp4_hbm,
                  deg_sh, row_v, ew_v, col_v, base_v, dbuf):
    cid = lax.axis_index("c")
    sid = lax.axis_index("s")
    w = cid * NSUB + sid
    off = w * TILE_EA
    pltpu.sync_copy(row_hbm.at[pl.ds(off, TILE_EA)], row_v)
    pltpu.sync_copy(ew_hbm.at[pl.ds(off, TILE_EA)], ew_v)
    pltpu.sync_copy(cola_hbm.at[w], col_v)
    pltpu.sync_copy(zflat_hbm.at[pl.ds(sid * (NT // NSUB), NT // NSUB)],
                    deg_sh.at[pl.ds(sid * (NT // NSUB), NT // NSUB)])
    plsc.subcore_barrier()

    def scat(j, _):
        pltpu.sync_copy(ew_v.at[pl.ds(j * 128, 128)],
                        deg_sh.at[col_v.at[j]], add=True)
        return 0
    lax.fori_loop(0, NBLK_A, scat, 0)

    def remap(i, _):
        sl = pl.ds(i * 16, 16)
        m = ew_v[sl] > 0.0
        base_v[sl] = jnp.where(m, row_v[sl], jnp.full((16,), ZROW, jnp.int32))
        return 0
    lax.fori_loop(0, TILE_EA // 16, remap, 0)
    for v in range(4):
        pltpu.sync_copy(base_v, rowp4_hbm.at[pl.ds(v * EPAD + off, TILE_EA)])
        if v < 3:
            def bump(i, _):
                sl = pl.ds(i * 16, 16)
                base_v[sl] = base_v[sl] + NT
                return 0
            lax.fori_loop(0, TILE_EA // 16, bump, 0)

    plsc.subcore_barrier()
    sl = pl.ds(sid * (NT // NSUB), NT // NSUB)
    pltpu.sync_copy(deg_sh.at[sl], dbuf)
    pltpu.sync_copy(dbuf, degp_hbm.at[pl.ds(cid * NT + sid * (NT // NSUB),
                                            NT // NSUB)])


def _sc_prep(rowf, ewf, col_a, zflat):
    return pl.kernel(
        _sc_prep_body,
        out_type=[jax.ShapeDtypeStruct((NCORE * NT,), jnp.float32),
                  jax.ShapeDtypeStruct((4 * EPAD,), jnp.int32)],
        mesh=_sc_mesh(),
        scratch_types=[
            pltpu.MemorySpace.VMEM_SHARED((NT,), jnp.float32),
            pltpu.MemorySpace.VMEM((TILE_EA,), jnp.int32),
            pltpu.MemorySpace.VMEM((TILE_EA,), jnp.float32),
            pltpu.MemorySpace.VMEM((NBLK_A, 128), jnp.int32),
            pltpu.MemorySpace.VMEM((TILE_EA,), jnp.int32),
            pltpu.MemorySpace.VMEM((NT // NSUB,), jnp.float32),
        ],
    )(rowf, ewf, col_a, zflat)


def _sc_agg_body(h_hbm, rowp_hbm, col_hbm, zrows_hbm,
                 agg_hbm,
                 acc_sh, row_v, col_v, rbuf, sem0, sem1):
    cid = lax.axis_index("c")
    sid = lax.axis_index("s")
    rows_per_tile = NT // NSUB
    sems = (sem0, sem1)
    for cc in range(2):
        c = 2 * cid + cc
        for k in range(rows_per_tile // 128):
            pltpu.sync_copy(zrows_hbm,
                            acc_sh.at[pl.ds(sid * rows_per_tile + k * 128, 128)])
        plsc.subcore_barrier()

        def gather_start(j, b):
            pltpu.async_copy(h_hbm.at[row_v.at[j]], rbuf.at[b], sems[b])

        def gather_wait(j, b):
            pltpu.make_async_copy(h_hbm.at[row_v.at[j]], rbuf.at[b],
                                  sems[b]).wait()

        def scat(j, b):
            pltpu.sync_copy(rbuf.at[b], acc_sh.at[col_v.at[j]], add=True)

        for hh in range(2):
            pltpu.sync_copy(rowp_hbm.at[c, sid, hh], row_v)
            pltpu.sync_copy(col_hbm.at[sid, hh], col_v)
            gather_start(0, 0)

            def step(jj, _):
                j0 = 2 * jj
                gather_start(j0 + 1, 1)
                gather_wait(j0, 0)
                scat(j0, 0)

                @pl.when(jj < NHALF // 2 - 1)
                def _():
                    gather_start(j0 + 2, 0)
                gather_wait(j0 + 1, 1)
                scat(j0 + 1, 1)
                return 0
            lax.fori_loop(0, NHALF // 2, step, 0)
        plsc.subcore_barrier()
        for k in range(rows_per_tile // KE):
            sl = pl.ds(sid * rows_per_tile + k * KE, KE)
            pltpu.sync_copy(acc_sh.at[sl], rbuf.at[0])
            pltpu.sync_copy(rbuf.at[0], agg_hbm.at[c, sl])
        plsc.subcore_barrier()


def _sc_agg(hflat, rowp4r, col3, zrows):
    return pl.kernel(
        _sc_agg_body,
        out_type=jax.ShapeDtypeStruct((4, NT, 128), jnp.float32),
        mesh=_sc_mesh(),
        scratch_types=[
            pltpu.MemorySpace.VMEM_SHARED((NT, 128), jnp.float32),
            pltpu.MemorySpace.VMEM((NHALF, 128), jnp.int32),
            pltpu.MemorySpace.VMEM((NHALF, 128), jnp.int32),
            pltpu.MemorySpace.VMEM((2, KE, 128), jnp.float32),
            pltpu.SemaphoreType.DMA,
            pltpu.SemaphoreType.DMA,
        ],
    )(hflat, rowp4r, col3, zrows)


def _tc_prep_body(x_ref, degp_ref, s1_ref, s2_ref, dinv_ref):
    x = x_ref[...]
    s1_ref[...] = jnp.sum(x, axis=0, keepdims=True)
    s2_ref[...] = jnp.sum(x * x, axis=0, keepdims=True)
    deg = degp_ref[0] + degp_ref[1]
    rid = (lax.broadcasted_iota(jnp.int32, (NT // 128, 128), 0) * 128
           + lax.broadcasted_iota(jnp.int32, (NT // 128, 128), 1))
    valid = (rid < N) & (deg > 0.0)
    dinv_ref[...] = jnp.where(valid, lax.rsqrt(jnp.maximum(deg, 1e-12)), 0.0)


def _tc_prep(xp, degp2):
    return pl.pallas_call(
        _tc_prep_body,
        out_shape=[jax.ShapeDtypeStruct((1, 128), jnp.float32),
                   jax.ShapeDtypeStruct((1, 128), jnp.float32),
                   jax.ShapeDtypeStruct((NT // 128, 128), jnp.float32)],
    )(xp, degp2)


def _bn_affine(s1, s2, w, b):
    mean = s1 / N
    var = s2 / N - mean * mean
    alpha = w * lax.rsqrt(var + EPS)
    beta = b - mean * alpha
    return alpha, beta


def _tc_mm1_body(x_ref, s1_ref, s2_ref, w_ref, b_ref, w1_ref, dinv_ref,
                 out_ref):
    alpha, beta = _bn_affine(s1_ref[...], s2_ref[...], w_ref[...], b_ref[...])
    xn = x_ref[...] * alpha + beta
    h = jnp.dot(xn, w1_ref[...], preferred_element_type=jnp.float32)
    h = h * dinv_ref[...]
    for c in range(4):
        out_ref[c] = h[:, c * 128:(c + 1) * 128]


def _tc_mm1(xp, s1x, s2x, bn1wp, bn1bp, w1p, dinv_c):
    g = NT // 128
    return pl.pallas_call(
        _tc_mm1_body,
        grid=(g,),
        in_specs=[
            pl.BlockSpec((128, 128), lambda i: (i, 0)),
            pl.BlockSpec((1, 128), lambda i: (0, 0)),
            pl.BlockSpec((1, 128), lambda i: (0, 0)),
            pl.BlockSpec((1, 128), lambda i: (0, 0)),
            pl.BlockSpec((1, 128), lambda i: (0, 0)),
            pl.BlockSpec((128, 512), lambda i: (0, 0)),
            pl.BlockSpec((128, 1), lambda i: (i, 0)),
        ],
        out_specs=pl.BlockSpec((4, 128, 128), lambda i: (0, i, 0)),
        out_shape=jax.ShapeDtypeStruct((4, NT, 128), jnp.float32),
    )(xp, s1x, s2x, bn1wp, bn1bp, w1p, dinv_c)


def _assemble(agg):
    return jnp.concatenate([agg[c] for c in range(4)], axis=1)


def _tc_stats1_body(agg_ref, dinv_ref, b_ref, s1_ref, s2_ref):
    i = pl.program_id(0)
    t = jnp.maximum(_assemble(agg_ref[...]) * dinv_ref[...] + b_ref[...], 0.0)
    rid = i * 128 + lax.broadcasted_iota(jnp.int32, (128, 1), 0)
    t = jnp.where(rid < N, t, 0.0)

    @pl.when(i == 0)
    def _():
        s1_ref[...] = jnp.zeros_like(s1_ref)
        s2_ref[...] = jnp.zeros_like(s2_ref)
    s1_ref[...] += jnp.sum(t, axis=0, keepdims=True)
    s2_ref[...] += jnp.sum(t * t, axis=0, keepdims=True)


def _tc_stats1(agg, dinv_c, br):
    g = NT // 128
    return pl.pallas_call(
        _tc_stats1_body,
        grid=(g,),
        in_specs=[
            pl.BlockSpec((4, 128, 128), lambda i: (0, i, 0)),
            pl.BlockSpec((128, 1), lambda i: (i, 0)),
            pl.BlockSpec((1, 512), lambda i: (0, 0)),
        ],
        out_specs=[pl.BlockSpec((1, 512), lambda i: (0, 0)),
                   pl.BlockSpec((1, 512), lambda i: (0, 0))],
        out_shape=[jax.ShapeDtypeStruct((1, 512), jnp.float32),
                   jax.ShapeDtypeStruct((1, 512), jnp.float32)],
    )(agg, dinv_c, br)


def _tc_mm2_body(agg_ref, dinv_ref, b1_ref, s1_ref, s2_ref, w_ref, b_ref,
                 w2_ref, out_ref):
    dinv = dinv_ref[...]
    t = jnp.maximum(_assemble(agg_ref[...]) * dinv + b1_ref[...], 0.0)
    alpha, beta = _bn_affine(s1_ref[...], s2_ref[...], w_ref[...], b_ref[...])
    tn = t * alpha + beta
    h = jnp.dot(tn, w2_ref[...], preferred_element_type=jnp.float32)
    h = h * dinv
    for c in range(4):
        out_ref[c] = h[:, c * 128:(c + 1) * 128]


def _tc_mm2(agg, dinv_c, b1r, s1, s2, bn2wr, bn2br, w2):
    g = NT // 128
    bs_full = lambda shape: pl.BlockSpec(shape, lambda i: tuple(0 for _ in shape))
    return pl.pallas_call(
        _tc_mm2_body,
        grid=(g,),
        in_specs=[
            pl.BlockSpec((4, 128, 128), lambda i: (0, i, 0)),
            pl.BlockSpec((128, 1), lambda i: (i, 0)),
            bs_full((1, 512)), bs_full((1, 512)), bs_full((1, 512)),
            bs_full((1, 512)), bs_full((1, 512)),
            bs_full((512, 512)),
        ],
        out_specs=pl.BlockSpec((4, 128, 128), lambda i: (0, i, 0)),
        out_shape=jax.ShapeDtypeStruct((4, NT, 128), jnp.float32),
    )(agg, dinv_c, b1r, s1, s2, bn2wr, bn2br, w2)


def _tc_stats2_body(agg_ref, dinv_ref, b_ref, s1_ref, s2_ref, mx_ref, mn_ref):
    i = pl.program_id(0)
    u = jnp.maximum(_assemble(agg_ref[...]) * dinv_ref[...] + b_ref[...], 0.0)
    rid = i * 128 + lax.broadcasted_iota(jnp.int32, (128, 1), 0)
    valid = rid < N
    uz = jnp.where(valid, u, 0.0)
    ux = jnp.where(valid, u, -3e38)
    un = jnp.where(valid, u, 3e38)

    @pl.when(i == 0)
    def _():
        s1_ref[...] = jnp.zeros_like(s1_ref)
        s2_ref[...] = jnp.zeros_like(s2_ref)
        mx_ref[...] = jnp.full_like(mx_ref, -3e38)
        mn_ref[...] = jnp.full_like(mn_ref, 3e38)
    s1_ref[...] += jnp.sum(uz, axis=0, keepdims=True)
    s2_ref[...] += jnp.sum(uz * uz, axis=0, keepdims=True)
    mx_ref[...] = jnp.maximum(mx_ref[...], jnp.max(ux, axis=0, keepdims=True))
    mn_ref[...] = jnp.minimum(mn_ref[...], jnp.min(un, axis=0, keepdims=True))


def _tc_stats2(agg, dinv_c, br):
    g = NT // 128
    return pl.pallas_call(
        _tc_stats2_body,
        grid=(g,),
        in_specs=[
            pl.BlockSpec((4, 128, 128), lambda i: (0, i, 0)),
            pl.BlockSpec((128, 1), lambda i: (i, 0)),
            pl.BlockSpec((1, 512), lambda i: (0, 0)),
        ],
        out_specs=[pl.BlockSpec((1, 512), lambda i: (0, 0))] * 4,
        out_shape=[jax.ShapeDtypeStruct((1, 512), jnp.float32)] * 4,
    )(agg, dinv_c, br)


def _tc_final_body(s1_ref, s2_ref, mx_ref, mn_ref, w3_ref, b3_ref,
                   fc1w_ref, fc1b_ref, fc2w_ref, fc2b_ref, out_ref):
    alpha, beta = _bn_affine(s1_ref[...], s2_ref[...], w3_ref[...], b3_ref[...])
    g0 = jnp.where(alpha >= 0.0, alpha * mx_ref[...], alpha * mn_ref[...]) + beta
    g1 = jnp.dot(g0, fc1w_ref[...], preferred_element_type=jnp.float32)
    g1 = jnp.maximum(g1 + fc1b_ref[...], 0.0)
    g2 = jnp.sum(g1 * fc2w_ref[...], axis=1, keepdims=True) + fc2b_ref[...]
    out_ref[...] = jax.nn.sigmoid(g2)


def _tc_final(s1, s2, mx, mn, bn3wr, bn3br, fc1w, fc1br, fc2wr, fc2br):
    return pl.pallas_call(
        _tc_final_body,
        out_shape=jax.ShapeDtypeStruct((1, 1), jnp.float32),
    )(s1, s2, mx, mn, bn3wr, bn3br, fc1w, fc1br, fc2wr, fc2br)


def kernel(x, edge_index, bn1_w, bn1_b, conv1_W, conv1_b, bn2_w, bn2_b,
           conv2_W, conv2_b, bn3_w, bn3_b, fc1_W, fc1_b, fc2_W, fc2_b):
    f32 = jnp.float32
    i32 = jnp.int32
    mask = jax.random.uniform(jax.random.key(42), (E,)) > DROP_PROB
    keep = jnp.argsort(jnp.logical_not(mask), stable=True)[:EKEEP]
    loop = jnp.arange(N, dtype=i32)
    row = jnp.concatenate([edge_index[0].astype(i32)[keep], loop])
    col = jnp.concatenate([edge_index[1].astype(i32)[keep], loop])
    pad = EPAD - E1
    rowf = jnp.pad(row, (0, pad))
    colf = jnp.pad(col, (0, pad), constant_values=ZROW)
    ewf = (jnp.arange(EPAD, dtype=i32) < E1).astype(f32)
    col_a = colf.reshape(32, NBLK_A, 128)
    col3 = colf.reshape(16, 2, NHALF, 128)
    zflat = jnp.zeros((NT,), f32)
    zrows = jnp.zeros((128, 128), f32)

    degp, rowp4 = _sc_prep(rowf, ewf, col_a, zflat)
    rowp4r = rowp4.reshape(4, 16, 2, NHALF, 128)

    xp = jnp.pad(x, ((0, NT - N), (0, 128 - x.shape[1])))
    w1p = jnp.pad(conv1_W, ((0, 128 - conv1_W.shape[0]), (0, 0)))
    bn1wp = jnp.pad(bn1_w, (0, 128 - bn1_w.shape[0])).reshape(1, 128)
    bn1bp = jnp.pad(bn1_b, (0, 128 - bn1_b.shape[0])).reshape(1, 128)
    s1x, s2x, dinv2d = _tc_prep(xp, degp.reshape(2, NT // 128, 128))
    dinv_c = dinv2d.reshape(NT, 1)

    b1r = conv1_b.reshape(1, 512)
    b2r = conv2_b.reshape(1, 512)
    bn2wr = bn2_w.reshape(1, 512)
    bn2br = bn2_b.reshape(1, 512)
    bn3wr = bn3_w.reshape(1, 512)
    bn3br = bn3_b.reshape(1, 512)
    fc1br = fc1_b.reshape(1, 512)
    fc2wr = fc2_W.reshape(1, 512)
    fc2br = fc2_b.reshape(1, 1)

    h1 = _tc_mm1(xp, s1x, s2x, bn1wp, bn1bp, w1p, dinv_c)
    agg1 = _sc_agg(h1.reshape(4 * NT, 128), rowp4r, col3, zrows)
    s1, s2 = _tc_stats1(agg1, dinv_c, b1r)
    h2 = _tc_mm2(agg1, dinv_c, b1r, s1, s2, bn2wr, bn2br, conv2_W)
    agg2 = _sc_agg(h2.reshape(4 * NT, 128), rowp4r, col3, zrows)
    s1b, s2b, mx, mn = _tc_stats2(agg2, dinv_c, b2r)
    return _tc_final(s1b, s2b, mx, mn, bn3wr, bn3br, fc1_W, fc1br,
                     fc2wr, fc2br)


# speedup vs baseline: 9.3428x; 2.6440x over previous
import functools

import jax
import jax.numpy as jnp
import numpy as np
from jax import lax
from jax.experimental import pallas as pl
from jax.experimental.pallas import tpu as pltpu
from jax.experimental.pallas import tpu_sc as plsc

N = 10000
E = 160000
EPS = 1e-5
DROP_PROB = 0.2

NT = 10240
EKEEP = 127863
E1 = EKEEP + N
EPAD = 139264
NBLK = 68
NHALF = 34
KE = 128
NBLK_A = 34
TILE_E = EPAD // 16
TILE_EA = EPAD // 32
ZROW = N
NSUB = 16
NCORE = 2


ER = E + N + 8


def _np_rotl(x, d):
    return ((x << np.uint32(d)) | (x >> np.uint32(32 - d))).astype(np.uint32)


def _np_threefry2x32(k1, k2, x0, x1):
    ks = [np.uint32(k1), np.uint32(k2),
          np.uint32(0x1BD11BDA) ^ np.uint32(k1) ^ np.uint32(k2)]
    rots = [(13, 15, 26, 6), (17, 29, 16, 24)]
    x0 = (x0 + ks[0]).astype(np.uint32)
    x1 = (x1 + ks[1]).astype(np.uint32)
    for i in range(5):
        for r in rots[i % 2]:
            x0 = (x0 + x1).astype(np.uint32)
            x1 = _np_rotl(x1, r) ^ x0
        x0 = (x0 + ks[(i + 1) % 3]).astype(np.uint32)
        x1 = (x1 + ks[(i + 2) % 3] + np.uint32(i + 1)).astype(np.uint32)
    return x0, x1


def _keep_pad() -> np.ndarray:
    x0, x1 = _np_threefry2x32(0, 42, np.zeros(E, np.uint32),
                              np.arange(E, dtype=np.uint32))
    bits = x0 ^ x1
    u = ((bits >> np.uint32(9)) | np.uint32(0x3F800000)).view(np.float32) \
        - np.float32(1.0)
    keep = np.nonzero(u > np.float32(DROP_PROB))[0].astype(np.int32)
    assert keep.size == EKEEP, keep.size
    sl = E + np.arange(N, dtype=np.int32)
    padv = np.full(EPAD - E1, E + N, np.int32)
    return np.concatenate([keep, sl, padv])


KEEP_PAD = _keep_pad()


def _sc_mesh():
    return plsc.VectorSubcoreMesh(
        core_axis_name="c", subcore_axis_name="s",
        num_cores=NCORE, num_subcores=NSUB)


def _sc_prep_body(er_hbm, ec_hbm, keep_hbm, ew_hbm, zflat_hbm,
                  degp_hbm, rowp4_hbm, colp_hbm,
                  deg_sh, keep_v, row_v, col_v, ew_v, dbuf, gsem):
    cid = lax.axis_index("c")
    sid = lax.axis_index("s")
    w = cid * NSUB + sid
    off = w * TILE_EA
    pltpu.sync_copy(keep_hbm.at[w], keep_v)
    pltpu.sync_copy(ew_hbm.at[pl.ds(off, TILE_EA)], ew_v)
    pltpu.sync_copy(zflat_hbm.at[pl.ds(sid * (NT // NSUB), NT // NSUB)],
                    deg_sh.at[pl.ds(sid * (NT // NSUB), NT // NSUB)])
    for j in range(NBLK_A):
        pltpu.async_copy(er_hbm.at[keep_v.at[j]], row_v.at[j], gsem)
        pltpu.async_copy(ec_hbm.at[keep_v.at[j]], col_v.at[j], gsem)
    for j in range(NBLK_A):
        pltpu.make_async_copy(er_hbm.at[keep_v.at[j]], row_v.at[j],
                              gsem).wait()
        pltpu.make_async_copy(ec_hbm.at[keep_v.at[j]], col_v.at[j],
                              gsem).wait()
    plsc.subcore_barrier()

    def scat(j, _):
        pltpu.sync_copy(ew_v.at[pl.ds(j * 128, 128)],
                        deg_sh.at[col_v.at[j]], add=True)
        return 0
    lax.fori_loop(0, NBLK_A, scat, 0)

    def remap(i, _):
        r = i // 8
        sl = pl.ds((i % 8) * 16, 16)
        m = ew_v[pl.ds(i * 16, 16)] > 0.0
        row_v[r, sl] = jnp.where(m, row_v[r, sl],
                                 jnp.full((16,), ZROW, jnp.int32))
        return 0
    lax.fori_loop(0, TILE_EA // 16, remap, 0)
    pltpu.sync_copy(col_v, colp_hbm.at[w])
    for v in range(4):
        pltpu.sync_copy(row_v, rowp4_hbm.at[v, w])
        if v < 3:
            def bump(i, _):
                r = i // 8
                sl = pl.ds((i % 8) * 16, 16)
                row_v[r, sl] = row_v[r, sl] + NT
                return 0
            lax.fori_loop(0, TILE_EA // 16, bump, 0)

    plsc.subcore_barrier()
    sl = pl.ds(sid * (NT // NSUB), NT // NSUB)
    pltpu.sync_copy(deg_sh.at[sl], dbuf)
    pltpu.sync_copy(dbuf, degp_hbm.at[pl.ds(cid * NT + sid * (NT // NSUB),
                                            NT // NSUB)])


def _sc_prep(er, ec, keep3, ewf, zflat):
    return pl.kernel(
        _sc_prep_body,
        out_type=[jax.ShapeDtypeStruct((NCORE * NT,), jnp.float32),
                  jax.ShapeDtypeStruct((4, 32, NBLK_A, 128), jnp.int32),
                  jax.ShapeDtypeStruct((32, NBLK_A, 128), jnp.int32)],
        mesh=_sc_mesh(),
        scratch_types=[
            pltpu.MemorySpace.VMEM_SHARED((NT,), jnp.float32),
            pltpu.MemorySpace.VMEM((NBLK_A, 128), jnp.int32),
            pltpu.MemorySpace.VMEM((NBLK_A, 128), jnp.int32),
            pltpu.MemorySpace.VMEM((NBLK_A, 128), jnp.int32),
            pltpu.MemorySpace.VMEM((TILE_EA,), jnp.float32),
            pltpu.MemorySpace.VMEM((NT // NSUB,), jnp.float32),
            pltpu.SemaphoreType.DMA,
        ],
    )(er, ec, keep3, ewf, zflat)


def _sc_agg_body(h_hbm, rowp_hbm, col_hbm, zrows_hbm,
                 agg_hbm,
                 acc_sh, row_v, col_v, rbuf, sem0, sem1):
    cid = lax.axis_index("c")
    sid = lax.axis_index("s")
    rows_per_tile = NT // NSUB
    sems = (sem0, sem1)
    for cc in range(2):
        c = 2 * cid + cc
        for k in range(rows_per_tile // 128):
            pltpu.sync_copy(zrows_hbm,
                            acc_sh.at[pl.ds(sid * rows_per_tile + k * 128, 128)])
        plsc.subcore_barrier()

        def gather_start(j, b):
            pltpu.async_copy(h_hbm.at[row_v.at[j]], rbuf.at[b], sems[b])

        def gather_wait(j, b):
            pltpu.make_async_copy(h_hbm.at[row_v.at[j]], rbuf.at[b],
                                  sems[b]).wait()

        def scat(j, b):
            pltpu.sync_copy(rbuf.at[b], acc_sh.at[col_v.at[j]], add=True)

        for hh in range(2):
            pltpu.sync_copy(rowp_hbm.at[c, sid, hh], row_v)
            pltpu.sync_copy(col_hbm.at[sid, hh], col_v)
            gather_start(0, 0)

            def step(jj, _):
                j0 = 2 * jj
                gather_start(j0 + 1, 1)
                gather_wait(j0, 0)
                scat(j0, 0)

                @pl.when(jj < NHALF // 2 - 1)
                def _():
                    gather_start(j0 + 2, 0)
                gather_wait(j0 + 1, 1)
                scat(j0 + 1, 1)
                return 0
            lax.fori_loop(0, NHALF // 2, step, 0)
        plsc.subcore_barrier()
        for k in range(rows_per_tile // KE):
            sl = pl.ds(sid * rows_per_tile + k * KE, KE)
            pltpu.sync_copy(acc_sh.at[sl], rbuf.at[0])
            pltpu.sync_copy(rbuf.at[0], agg_hbm.at[c, sl])
        plsc.subcore_barrier()


def _sc_agg(hflat, rowp4r, col3, zrows):
    return pl.kernel(
        _sc_agg_body,
        out_type=jax.ShapeDtypeStruct((4, NT, 128), jnp.float32),
        mesh=_sc_mesh(),
        scratch_types=[
            pltpu.MemorySpace.VMEM_SHARED((NT, 128), jnp.float32),
            pltpu.MemorySpace.VMEM((NHALF, 128), jnp.int32),
            pltpu.MemorySpace.VMEM((NHALF, 128), jnp.int32),
            pltpu.MemorySpace.VMEM((2, KE, 128), jnp.float32),
            pltpu.SemaphoreType.DMA,
            pltpu.SemaphoreType.DMA,
        ],
    )(hflat, rowp4r, col3, zrows)


def _tc_prep_body(x_ref, degp_ref, s1_ref, s2_ref, dinv_ref):
    x = x_ref[...]
    s1_ref[...] = jnp.sum(x, axis=0, keepdims=True)
    s2_ref[...] = jnp.sum(x * x, axis=0, keepdims=True)
    deg = degp_ref[0] + degp_ref[1]
    rid = (lax.broadcasted_iota(jnp.int32, (NT // 128, 128), 0) * 128
           + lax.broadcasted_iota(jnp.int32, (NT // 128, 128), 1))
    valid = (rid < N) & (deg > 0.0)
    dinv_ref[...] = jnp.where(valid, lax.rsqrt(jnp.maximum(deg, 1e-12)), 0.0)


def _tc_prep(xp, degp2):
    return pl.pallas_call(
        _tc_prep_body,
        out_shape=[jax.ShapeDtypeStruct((1, 128), jnp.float32),
                   jax.ShapeDtypeStruct((1, 128), jnp.float32),
                   jax.ShapeDtypeStruct((NT // 128, 128), jnp.float32)],
    )(xp, degp2)


def _bn_affine(s1, s2, w, b):
    mean = s1 / N
    var = s2 / N - mean * mean
    alpha = w * lax.rsqrt(var + EPS)
    beta = b - mean * alpha
    return alpha, beta


def _tc_mm1_body(x_ref, s1_ref, s2_ref, w_ref, b_ref, w1_ref, dinv_ref,
                 out_ref):
    alpha, beta = _bn_affine(s1_ref[...], s2_ref[...], w_ref[...], b_ref[...])
    xn = x_ref[...] * alpha + beta
    h = jnp.dot(xn, w1_ref[...], preferred_element_type=jnp.float32)
    h = h * dinv_ref[...]
    for c in range(4):
        out_ref[c] = h[:, c * 128:(c + 1) * 128]


def _tc_mm1(xp, s1x, s2x, bn1wp, bn1bp, w1p, dinv_c):
    g = NT // 128
    return pl.pallas_call(
        _tc_mm1_body,
        grid=(g,),
        in_specs=[
            pl.BlockSpec((128, 128), lambda i: (i, 0)),
            pl.BlockSpec((1, 128), lambda i: (0, 0)),
            pl.BlockSpec((1, 128), lambda i: (0, 0)),
            pl.BlockSpec((1, 128), lambda i: (0, 0)),
            pl.BlockSpec((1, 128), lambda i: (0, 0)),
            pl.BlockSpec((128, 512), lambda i: (0, 0)),
            pl.BlockSpec((128, 1), lambda i: (i, 0)),
        ],
        out_specs=pl.BlockSpec((4, 128, 128), lambda i: (0, i, 0)),
        out_shape=jax.ShapeDtypeStruct((4, NT, 128), jnp.float32),
    )(xp, s1x, s2x, bn1wp, bn1bp, w1p, dinv_c)


def _assemble(agg):
    return jnp.concatenate([agg[c] for c in range(4)], axis=1)


def _tc_stats1_body(agg_ref, dinv_ref, b_ref, s1_ref, s2_ref):
    i = pl.program_id(0)
    t = jnp.maximum(_assemble(agg_ref[...]) * dinv_ref[...] + b_ref[...], 0.0)
    rid = i * 128 + lax.broadcasted_iota(jnp.int32, (128, 1), 0)
    t = jnp.where(rid < N, t, 0.0)

    @pl.when(i == 0)
    def _():
        s1_ref[...] = jnp.zeros_like(s1_ref)
        s2_ref[...] = jnp.zeros_like(s2_ref)
    s1_ref[...] += jnp.sum(t, axis=0, keepdims=True)
    s2_ref[...] += jnp.sum(t * t, axis=0, keepdims=True)


def _tc_stats1(agg, dinv_c, br):
    g = NT // 128
    return pl.pallas_call(
        _tc_stats1_body,
        grid=(g,),
        in_specs=[
            pl.BlockSpec((4, 128, 128), lambda i: (0, i, 0)),
            pl.BlockSpec((128, 1), lambda i: (i, 0)),
            pl.BlockSpec((1, 512), lambda i: (0, 0)),
        ],
        out_specs=[pl.BlockSpec((1, 512), lambda i: (0, 0)),
                   pl.BlockSpec((1, 512), lambda i: (0, 0))],
        out_shape=[jax.ShapeDtypeStruct((1, 512), jnp.float32),
                   jax.ShapeDtypeStruct((1, 512), jnp.float32)],
    )(agg, dinv_c, br)


def _tc_mm2_body(agg_ref, dinv_ref, b1_ref, s1_ref, s2_ref, w_ref, b_ref,
                 w2_ref, out_ref):
    dinv = dinv_ref[...]
    t = jnp.maximum(_assemble(agg_ref[...]) * dinv + b1_ref[...], 0.0)
    alpha, beta = _bn_affine(s1_ref[...], s2_ref[...], w_ref[...], b_ref[...])
    tn = t * alpha + beta
    h = jnp.dot(tn, w2_ref[...], preferred_element_type=jnp.float32)
    h = h * dinv
    for c in range(4):
        out_ref[c] = h[:, c * 128:(c + 1) * 128]


def _tc_mm2(agg, dinv_c, b1r, s1, s2, bn2wr, bn2br, w2):
    g = NT // 128
    bs_full = lambda shape: pl.BlockSpec(shape, lambda i: tuple(0 for _ in shape))
    return pl.pallas_call(
        _tc_mm2_body,
        grid=(g,),
        in_specs=[
            pl.BlockSpec((4, 128, 128), lambda i: (0, i, 0)),
            pl.BlockSpec((128, 1), lambda i: (i, 0)),
            bs_full((1, 512)), bs_full((1, 512)), bs_full((1, 512)),
            bs_full((1, 512)), bs_full((1, 512)),
            bs_full((512, 512)),
        ],
        out_specs=pl.BlockSpec((4, 128, 128), lambda i: (0, i, 0)),
        out_shape=jax.ShapeDtypeStruct((4, NT, 128), jnp.float32),
    )(agg, dinv_c, b1r, s1, s2, bn2wr, bn2br, w2)


def _tc_stats2_body(agg_ref, dinv_ref, b_ref, s1_ref, s2_ref, mx_ref, mn_ref):
    i = pl.program_id(0)
    u = jnp.maximum(_assemble(agg_ref[...]) * dinv_ref[...] + b_ref[...], 0.0)
    rid = i * 128 + lax.broadcasted_iota(jnp.int32, (128, 1), 0)
    valid = rid < N
    uz = jnp.where(valid, u, 0.0)
    ux = jnp.where(valid, u, -3e38)
    un = jnp.where(valid, u, 3e38)

    @pl.when(i == 0)
    def _():
        s1_ref[...] = jnp.zeros_like(s1_ref)
        s2_ref[...] = jnp.zeros_like(s2_ref)
        mx_ref[...] = jnp.full_like(mx_ref, -3e38)
        mn_ref[...] = jnp.full_like(mn_ref, 3e38)
    s1_ref[...] += jnp.sum(uz, axis=0, keepdims=True)
    s2_ref[...] += jnp.sum(uz * uz, axis=0, keepdims=True)
    mx_ref[...] = jnp.maximum(mx_ref[...], jnp.max(ux, axis=0, keepdims=True))
    mn_ref[...] = jnp.minimum(mn_ref[...], jnp.min(un, axis=0, keepdims=True))


def _tc_stats2(agg, dinv_c, br):
    g = NT // 128
    return pl.pallas_call(
        _tc_stats2_body,
        grid=(g,),
        in_specs=[
            pl.BlockSpec((4, 128, 128), lambda i: (0, i, 0)),
            pl.BlockSpec((128, 1), lambda i: (i, 0)),
            pl.BlockSpec((1, 512), lambda i: (0, 0)),
        ],
        out_specs=[pl.BlockSpec((1, 512), lambda i: (0, 0))] * 4,
        out_shape=[jax.ShapeDtypeStruct((1, 512), jnp.float32)] * 4,
    )(agg, dinv_c, br)


def _tc_final_body(s1_ref, s2_ref, mx_ref, mn_ref, w3_ref, b3_ref,
                   fc1w_ref, fc1b_ref, fc2w_ref, fc2b_ref, out_ref):
    alpha, beta = _bn_affine(s1_ref[...], s2_ref[...], w3_ref[...], b3_ref[...])
    g0 = jnp.where(alpha >= 0.0, alpha * mx_ref[...], alpha * mn_ref[...]) + beta
    g1 = jnp.dot(g0, fc1w_ref[...], preferred_element_type=jnp.float32)
    g1 = jnp.maximum(g1 + fc1b_ref[...], 0.0)
    g2 = jnp.sum(g1 * fc2w_ref[...], axis=1, keepdims=True) + fc2b_ref[...]
    out_ref[...] = jax.nn.sigmoid(g2)


def _tc_final(s1, s2, mx, mn, bn3wr, bn3br, fc1w, fc1br, fc2wr, fc2br):
    return pl.pallas_call(
        _tc_final_body,
        out_shape=jax.ShapeDtypeStruct((1, 1), jnp.float32),
    )(s1, s2, mx, mn, bn3wr, bn3br, fc1w, fc1br, fc2wr, fc2br)


def kernel(x, edge_index, bn1_w, bn1_b, conv1_W, conv1_b, bn2_w, bn2_b,
           conv2_W, conv2_b, bn3_w, bn3_b, fc1_W, fc1_b, fc2_W, fc2_b):
    f32 = jnp.float32
    i32 = jnp.int32
    loop = jnp.arange(N, dtype=i32)
    dump = jnp.full((8,), ZROW, i32)
    er = jnp.concatenate([edge_index[0].astype(i32), loop, dump])
    ec = jnp.concatenate([edge_index[1].astype(i32), loop, dump])
    keep3 = jnp.asarray(KEEP_PAD).reshape(32, NBLK_A, 128)
    ewf = (jnp.arange(EPAD, dtype=i32) < E1).astype(f32)
    zflat = jnp.zeros((NT,), f32)
    zrows = jnp.zeros((128, 128), f32)

    degp, rowp4, colp = _sc_prep(er, ec, keep3, ewf, zflat)
    rowp4r = rowp4.reshape(4, 16, 2, NHALF, 128)
    col3 = colp.reshape(16, 2, NHALF, 128)

    xp = jnp.pad(x, ((0, NT - N), (0, 128 - x.shape[1])))
    w1p = jnp.pad(conv1_W, ((0, 128 - conv1_W.shape[0]), (0, 0)))
    bn1wp = jnp.pad(bn1_w, (0, 128 - bn1_w.shape[0])).reshape(1, 128)
    bn1bp = jnp.pad(bn1_b, (0, 128 - bn1_b.shape[0])).reshape(1, 128)
    s1x, s2x, dinv2d = _tc_prep(xp, degp.reshape(2, NT // 128, 128))
    dinv_c = dinv2d.reshape(NT, 1)

    b1r = conv1_b.reshape(1, 512)
    b2r = conv2_b.reshape(1, 512)
    bn2wr = bn2_w.reshape(1, 512)
    bn2br = bn2_b.reshape(1, 512)
    bn3wr = bn3_w.reshape(1, 512)
    bn3br = bn3_b.reshape(1, 512)
    fc1br = fc1_b.reshape(1, 512)
    fc2wr = fc2_W.reshape(1, 512)
    fc2br = fc2_b.reshape(1, 1)

    h1 = _tc_mm1(xp, s1x, s2x, bn1wp, bn1bp, w1p, dinv_c)
    agg1 = _sc_agg(h1.reshape(4 * NT, 128), rowp4r, col3, zrows)
    s1, s2 = _tc_stats1(agg1, dinv_c, b1r)
    h2 = _tc_mm2(agg1, dinv_c, b1r, s1, s2, bn2wr, bn2br, conv2_W)
    agg2 = _sc_agg(h2.reshape(4 * NT, 128), rowp4r, col3, zrows)
    s1b, s2b, mx, mn = _tc_stats2(agg2, dinv_c, b2r)
    return _tc_final(s1b, s2b, mx, mn, bn3wr, bn3br, fc1_W, fc1br,
                     fc2wr, fc2br)


# speedup vs baseline: 11.9636x; 1.2805x over previous
import functools

import jax
import jax.numpy as jnp
import numpy as np
from jax import lax
from jax.experimental import pallas as pl
from jax.experimental.pallas import tpu as pltpu
from jax.experimental.pallas import tpu_sc as plsc

N = 10000
E = 160000
EPS = 1e-5
DROP_PROB = 0.2

NT = 10240
EKEEP = 127863
E1 = EKEEP + N
EPAD = 139264
NBLK = 68
NHALF = 34
KE = 128
NBLK_A = 34
TILE_E = EPAD // 16
TILE_EA = EPAD // 32
ZROW = N
NSUB = 16
NCORE = 2


ER = E + N + 8


def _np_rotl(x, d):
    return ((x << np.uint32(d)) | (x >> np.uint32(32 - d))).astype(np.uint32)


def _np_threefry2x32(k1, k2, x0, x1):
    ks = [np.uint32(k1), np.uint32(k2),
          np.uint32(0x1BD11BDA) ^ np.uint32(k1) ^ np.uint32(k2)]
    rots = [(13, 15, 26, 6), (17, 29, 16, 24)]
    x0 = (x0 + ks[0]).astype(np.uint32)
    x1 = (x1 + ks[1]).astype(np.uint32)
    for i in range(5):
        for r in rots[i % 2]:
            x0 = (x0 + x1).astype(np.uint32)
            x1 = _np_rotl(x1, r) ^ x0
        x0 = (x0 + ks[(i + 1) % 3]).astype(np.uint32)
        x1 = (x1 + ks[(i + 2) % 3] + np.uint32(i + 1)).astype(np.uint32)
    return x0, x1


def _keep_pad() -> np.ndarray:
    x0, x1 = _np_threefry2x32(0, 42, np.zeros(E, np.uint32),
                              np.arange(E, dtype=np.uint32))
    bits = x0 ^ x1
    u = ((bits >> np.uint32(9)) | np.uint32(0x3F800000)).view(np.float32) \
        - np.float32(1.0)
    keep = np.nonzero(u > np.float32(DROP_PROB))[0].astype(np.int32)
    assert keep.size == EKEEP, keep.size
    sl = E + np.arange(N, dtype=np.int32)
    padv = np.full(EPAD - E1, E + N, np.int32)
    return np.concatenate([keep, sl, padv])


KEEP_PAD = _keep_pad()


def _sc_mesh():
    return plsc.VectorSubcoreMesh(
        core_axis_name="c", subcore_axis_name="s",
        num_cores=NCORE, num_subcores=NSUB)


def _sc_prep_body(er_hbm, ec_hbm, keep_hbm, ew_hbm, zflat_hbm,
                  degp_hbm, rowp4_hbm, colp_hbm,
                  deg_sh, keep_v, row_v, col_v, ew_v, dbuf, gsem):
    cid = lax.axis_index("c")
    sid = lax.axis_index("s")
    w = cid * NSUB + sid
    off = w * TILE_EA
    pltpu.sync_copy(keep_hbm.at[w], keep_v)
    pltpu.sync_copy(ew_hbm.at[pl.ds(off, TILE_EA)], ew_v)
    pltpu.sync_copy(zflat_hbm.at[pl.ds(sid * (NT // NSUB), NT // NSUB)],
                    deg_sh.at[pl.ds(sid * (NT // NSUB), NT // NSUB)])
    for j in range(NBLK_A):
        pltpu.async_copy(er_hbm.at[keep_v.at[j]], row_v.at[j], gsem)
        pltpu.async_copy(ec_hbm.at[keep_v.at[j]], col_v.at[j], gsem)
    for j in range(NBLK_A):
        pltpu.make_async_copy(er_hbm.at[keep_v.at[j]], row_v.at[j],
                              gsem).wait()
        pltpu.make_async_copy(ec_hbm.at[keep_v.at[j]], col_v.at[j],
                              gsem).wait()
    plsc.subcore_barrier()

    def scat(j, _):
        pltpu.sync_copy(ew_v.at[pl.ds(j * 128, 128)],
                        deg_sh.at[col_v.at[j]], add=True)
        return 0
    lax.fori_loop(0, NBLK_A, scat, 0)

    def remap(i, _):
        r = i // 8
        sl = pl.ds((i % 8) * 16, 16)
        m = ew_v[pl.ds(i * 16, 16)] > 0.0
        row_v[r, sl] = jnp.where(m, row_v[r, sl],
                                 jnp.full((16,), ZROW, jnp.int32))
        return 0
    lax.fori_loop(0, TILE_EA // 16, remap, 0)
    pltpu.sync_copy(col_v, colp_hbm.at[w])
    for v in range(4):
        pltpu.sync_copy(row_v, rowp4_hbm.at[v, w])
        if v < 3:
            def bump(i, _):
                r = i // 8
                sl = pl.ds((i % 8) * 16, 16)
                row_v[r, sl] = row_v[r, sl] + NT
                return 0
            lax.fori_loop(0, TILE_EA // 16, bump, 0)

    plsc.subcore_barrier()
    sl = pl.ds(sid * (NT // NSUB), NT // NSUB)
    pltpu.sync_copy(deg_sh.at[sl], dbuf)
    pltpu.sync_copy(dbuf, degp_hbm.at[pl.ds(cid * NT + sid * (NT // NSUB),
                                            NT // NSUB)])


def _sc_prep(er, ec, keep3, ewf, zflat):
    return pl.kernel(
        _sc_prep_body,
        out_type=[jax.ShapeDtypeStruct((NCORE * NT,), jnp.float32),
                  jax.ShapeDtypeStruct((4, 32, NBLK_A, 128), jnp.int32),
                  jax.ShapeDtypeStruct((32, NBLK_A, 128), jnp.int32)],
        mesh=_sc_mesh(),
        scratch_types=[
            pltpu.MemorySpace.VMEM_SHARED((NT,), jnp.float32),
            pltpu.MemorySpace.VMEM((NBLK_A, 128), jnp.int32),
            pltpu.MemorySpace.VMEM((NBLK_A, 128), jnp.int32),
            pltpu.MemorySpace.VMEM((NBLK_A, 128), jnp.int32),
            pltpu.MemorySpace.VMEM((TILE_EA,), jnp.float32),
            pltpu.MemorySpace.VMEM((NT // NSUB,), jnp.float32),
            pltpu.SemaphoreType.DMA,
        ],
    )(er, ec, keep3, ewf, zflat)


def _sc_agg_body(h_hbm, rowp_hbm, col_hbm, zrows_hbm,
                 agg_hbm,
                 acc_sh, row_v, col_v, rbuf, sem0, sem1):
    cid = lax.axis_index("c")
    sid = lax.axis_index("s")
    rows_per_tile = NT // NSUB
    sems = (sem0, sem1)
    for cc in range(2):
        c = 2 * cid + cc
        for k in range(rows_per_tile // 128):
            pltpu.sync_copy(zrows_hbm,
                            acc_sh.at[pl.ds(sid * rows_per_tile + k * 128, 128)])
        plsc.subcore_barrier()

        def gather_start(j, b):
            pltpu.async_copy(h_hbm.at[row_v.at[j]], rbuf.at[b], sems[b])

        def gather_wait(j, b):
            pltpu.make_async_copy(h_hbm.at[row_v.at[j]], rbuf.at[b],
                                  sems[b]).wait()

        def scat(j, b):
            pltpu.sync_copy(rbuf.at[b], acc_sh.at[col_v.at[j]], add=True)

        for hh in range(2):
            pltpu.sync_copy(rowp_hbm.at[c, sid, hh], row_v)
            pltpu.sync_copy(col_hbm.at[sid, hh], col_v)
            gather_start(0, 0)

            def step(jj, _):
                j0 = 2 * jj
                gather_start(j0 + 1, 1)
                gather_wait(j0, 0)
                scat(j0, 0)

                @pl.when(jj < NHALF // 2 - 1)
                def _():
                    gather_start(j0 + 2, 0)
                gather_wait(j0 + 1, 1)
                scat(j0 + 1, 1)
                return 0
            lax.fori_loop(0, NHALF // 2, step, 0)
        plsc.subcore_barrier()
        for k in range(rows_per_tile // KE):
            sl = pl.ds(sid * rows_per_tile + k * KE, KE)
            pltpu.sync_copy(acc_sh.at[sl], rbuf.at[0])
            pltpu.sync_copy(rbuf.at[0], agg_hbm.at[c, sl])
        plsc.subcore_barrier()


def _sc_agg(hflat, rowp4r, col3, zrows):
    return pl.kernel(
        _sc_agg_body,
        out_type=jax.ShapeDtypeStruct((4, NT, 128), jnp.float32),
        mesh=_sc_mesh(),
        scratch_types=[
            pltpu.MemorySpace.VMEM_SHARED((NT, 128), jnp.float32),
            pltpu.MemorySpace.VMEM((NHALF, 128), jnp.int32),
            pltpu.MemorySpace.VMEM((NHALF, 128), jnp.int32),
            pltpu.MemorySpace.VMEM((2, KE, 128), jnp.float32),
            pltpu.SemaphoreType.DMA,
            pltpu.SemaphoreType.DMA,
        ],
    )(hflat, rowp4r, col3, zrows)


def _sc_aggx_body(xn_hbm, rowp_hbm, colp_hbm, zrows_hbm,
                  agg_hbm,
                  acc_sh, row_v, col_v, rbuf, sem0, sem1):
    cid = lax.axis_index("c")
    sid = lax.axis_index("s")
    w = cid * NSUB + sid
    rows_per_tile = NT // NSUB
    sems = (sem0, sem1)
    pltpu.sync_copy(rowp_hbm.at[0, w], row_v)
    pltpu.sync_copy(colp_hbm.at[w], col_v)
    for k in range(rows_per_tile // 128):
        pltpu.sync_copy(zrows_hbm,
                        acc_sh.at[pl.ds(sid * rows_per_tile + k * 128, 128)])
    plsc.subcore_barrier()

    def gather_start(j, b):
        pltpu.async_copy(xn_hbm.at[row_v.at[j]], rbuf.at[b], sems[b])

    def gather_wait(j, b):
        pltpu.make_async_copy(xn_hbm.at[row_v.at[j]], rbuf.at[b],
                              sems[b]).wait()

    def scat(j, b):
        pltpu.sync_copy(rbuf.at[b], acc_sh.at[col_v.at[j]], add=True)

    gather_start(0, 0)

    def step(jj, _):
        j0 = 2 * jj
        gather_start(j0 + 1, 1)
        gather_wait(j0, 0)
        scat(j0, 0)

        @pl.when(jj < NBLK_A // 2 - 1)
        def _():
            gather_start(j0 + 2, 0)
        gather_wait(j0 + 1, 1)
        scat(j0 + 1, 1)
        return 0
    lax.fori_loop(0, NBLK_A // 2, step, 0)
    plsc.subcore_barrier()
    for k in range(rows_per_tile // KE):
        sl = pl.ds(sid * rows_per_tile + k * KE, KE)
        pltpu.sync_copy(acc_sh.at[sl], rbuf.at[0])
        pltpu.sync_copy(rbuf.at[0], agg_hbm.at[cid, sl])


def _sc_agg16(xn, rowp4, colp, zrows):
    return pl.kernel(
        _sc_aggx_body,
        out_type=jax.ShapeDtypeStruct((NCORE, NT, 128), jnp.float32),
        mesh=_sc_mesh(),
        scratch_types=[
            pltpu.MemorySpace.VMEM_SHARED((NT, 128), jnp.float32),
            pltpu.MemorySpace.VMEM((NBLK_A, 128), jnp.int32),
            pltpu.MemorySpace.VMEM((NBLK_A, 128), jnp.int32),
            pltpu.MemorySpace.VMEM((2, KE, 128), jnp.float32),
            pltpu.SemaphoreType.DMA,
            pltpu.SemaphoreType.DMA,
        ],
    )(xn, rowp4, colp, zrows)


def _tc_prep_body(x_ref, degp_ref, s1_ref, s2_ref, dinv_ref):
    x = x_ref[...]
    s1_ref[...] = jnp.sum(x, axis=0, keepdims=True)
    s2_ref[...] = jnp.sum(x * x, axis=0, keepdims=True)
    deg = degp_ref[0] + degp_ref[1]
    rid = (lax.broadcasted_iota(jnp.int32, (NT // 128, 128), 0) * 128
           + lax.broadcasted_iota(jnp.int32, (NT // 128, 128), 1))
    valid = (rid < N) & (deg > 0.0)
    dinv_ref[...] = jnp.where(valid, lax.rsqrt(jnp.maximum(deg, 1e-12)), 0.0)


def _tc_prep(xp, degp2):
    return pl.pallas_call(
        _tc_prep_body,
        out_shape=[jax.ShapeDtypeStruct((1, 128), jnp.float32),
                   jax.ShapeDtypeStruct((1, 128), jnp.float32),
                   jax.ShapeDtypeStruct((NT // 128, 128), jnp.float32)],
    )(xp, degp2)


def _bn_affine(s1, s2, w, b):
    mean = s1 / N
    var = s2 / N - mean * mean
    alpha = w * lax.rsqrt(var + EPS)
    beta = b - mean * alpha
    return alpha, beta


def _tc_xn_body(x_ref, s1_ref, s2_ref, w_ref, b_ref, dinv_ref, out_ref):
    alpha, beta = _bn_affine(s1_ref[...], s2_ref[...], w_ref[...], b_ref[...])
    out_ref[...] = (x_ref[...] * alpha + beta) * dinv_ref[...]


def _tc_xn(xp, s1x, s2x, bn1wp, bn1bp, dinv_c):
    g = NT // 128
    return pl.pallas_call(
        _tc_xn_body,
        grid=(g,),
        in_specs=[
            pl.BlockSpec((128, 128), lambda i: (i, 0)),
            pl.BlockSpec((1, 128), lambda i: (0, 0)),
            pl.BlockSpec((1, 128), lambda i: (0, 0)),
            pl.BlockSpec((1, 128), lambda i: (0, 0)),
            pl.BlockSpec((1, 128), lambda i: (0, 0)),
            pl.BlockSpec((128, 1), lambda i: (i, 0)),
        ],
        out_specs=pl.BlockSpec((128, 128), lambda i: (i, 0)),
        out_shape=jax.ShapeDtypeStruct((NT, 128), jnp.float32),
    )(xp, s1x, s2x, bn1wp, bn1bp, dinv_c)


def _assemble(agg):
    return jnp.concatenate([agg[c] for c in range(4)], axis=1)


def _conv1_t(p_ref, dinv_ref, b1_ref, w1_ref):
    ax = p_ref[0] + p_ref[1]
    h = jnp.dot(ax, w1_ref[...], preferred_element_type=jnp.float32)
    return jnp.maximum(h * dinv_ref[...] + b1_ref[...], 0.0)


def _tc_stats1_body(p_ref, dinv_ref, b1_ref, w1_ref, s1_ref, s2_ref):
    i = pl.program_id(0)
    t = _conv1_t(p_ref, dinv_ref, b1_ref, w1_ref)
    rid = i * 128 + lax.broadcasted_iota(jnp.int32, (128, 1), 0)
    t = jnp.where(rid < N, t, 0.0)

    @pl.when(i == 0)
    def _():
        s1_ref[...] = jnp.zeros_like(s1_ref)
        s2_ref[...] = jnp.zeros_like(s2_ref)
    s1_ref[...] += jnp.sum(t, axis=0, keepdims=True)
    s2_ref[...] += jnp.sum(t * t, axis=0, keepdims=True)


def _tc_stats1(aggx, dinv_c, b1r, w1p):
    g = NT // 128
    return pl.pallas_call(
        _tc_stats1_body,
        grid=(g,),
        in_specs=[
            pl.BlockSpec((2, 128, 128), lambda i: (0, i, 0)),
            pl.BlockSpec((128, 1), lambda i: (i, 0)),
            pl.BlockSpec((1, 512), lambda i: (0, 0)),
            pl.BlockSpec((128, 512), lambda i: (0, 0)),
        ],
        out_specs=[pl.BlockSpec((1, 512), lambda i: (0, 0)),
                   pl.BlockSpec((1, 512), lambda i: (0, 0))],
        out_shape=[jax.ShapeDtypeStruct((1, 512), jnp.float32),
                   jax.ShapeDtypeStruct((1, 512), jnp.float32)],
    )(aggx, dinv_c, b1r, w1p)


def _tc_mm2_body(p_ref, dinv_ref, b1_ref, w1_ref, s1_ref, s2_ref, w_ref,
                 b_ref, w2_ref, out_ref):
    dinv = dinv_ref[...]
    t = _conv1_t(p_ref, dinv_ref, b1_ref, w1_ref)
    alpha, beta = _bn_affine(s1_ref[...], s2_ref[...], w_ref[...], b_ref[...])
    tn = t * alpha + beta
    h = jnp.dot(tn, w2_ref[...], preferred_element_type=jnp.float32)
    h = h * dinv
    for c in range(4):
        out_ref[c] = h[:, c * 128:(c + 1) * 128]


def _tc_mm2(aggx, dinv_c, b1r, w1p, s1, s2, bn2wr, bn2br, w2):
    g = NT // 128
    bs_full = lambda shape: pl.BlockSpec(shape, lambda i: tuple(0 for _ in shape))
    return pl.pallas_call(
        _tc_mm2_body,
        grid=(g,),
        in_specs=[
            pl.BlockSpec((2, 128, 128), lambda i: (0, i, 0)),
            pl.BlockSpec((128, 1), lambda i: (i, 0)),
            bs_full((1, 512)),
            bs_full((128, 512)),
            bs_full((1, 512)), bs_full((1, 512)),
            bs_full((1, 512)), bs_full((1, 512)),
            bs_full((512, 512)),
        ],
        out_specs=pl.BlockSpec((4, 128, 128), lambda i: (0, i, 0)),
        out_shape=jax.ShapeDtypeStruct((4, NT, 128), jnp.float32),
    )(aggx, dinv_c, b1r, w1p, s1, s2, bn2wr, bn2br, w2)


def _tc_stats2_body(agg_ref, dinv_ref, b_ref, s1_ref, s2_ref, mx_ref, mn_ref):
    i = pl.program_id(0)
    u = jnp.maximum(_assemble(agg_ref[...]) * dinv_ref[...] + b_ref[...], 0.0)
    rid = i * 128 + lax.broadcasted_iota(jnp.int32, (128, 1), 0)
    valid = rid < N
    uz = jnp.where(valid, u, 0.0)
    ux = jnp.where(valid, u, -3e38)
    un = jnp.where(valid, u, 3e38)

    @pl.when(i == 0)
    def _():
        s1_ref[...] = jnp.zeros_like(s1_ref)
        s2_ref[...] = jnp.zeros_like(s2_ref)
        mx_ref[...] = jnp.full_like(mx_ref, -3e38)
        mn_ref[...] = jnp.full_like(mn_ref, 3e38)
    s1_ref[...] += jnp.sum(uz, axis=0, keepdims=True)
    s2_ref[...] += jnp.sum(uz * uz, axis=0, keepdims=True)
    mx_ref[...] = jnp.maximum(mx_ref[...], jnp.max(ux, axis=0, keepdims=True))
    mn_ref[...] = jnp.minimum(mn_ref[...], jnp.min(un, axis=0, keepdims=True))


def _tc_stats2(agg, dinv_c, br):
    g = NT // 128
    return pl.pallas_call(
        _tc_stats2_body,
        grid=(g,),
        in_specs=[
            pl.BlockSpec((4, 128, 128), lambda i: (0, i, 0)),
            pl.BlockSpec((128, 1), lambda i: (i, 0)),
            pl.BlockSpec((1, 512), lambda i: (0, 0)),
        ],
        out_specs=[pl.BlockSpec((1, 512), lambda i: (0, 0))] * 4,
        out_shape=[jax.ShapeDtypeStruct((1, 512), jnp.float32)] * 4,
    )(agg, dinv_c, br)


def _tc_final_body(s1_ref, s2_ref, mx_ref, mn_ref, w3_ref, b3_ref,
                   fc1w_ref, fc1b_ref, fc2w_ref, fc2b_ref, out_ref):
    alpha, beta = _bn_affine(s1_ref[...], s2_ref[...], w3_ref[...], b3_ref[...])
    g0 = jnp.where(alpha >= 0.0, alpha * mx_ref[...], alpha * mn_ref[...]) + beta
    g1 = jnp.dot(g0, fc1w_ref[...], preferred_element_type=jnp.float32)
    g1 = jnp.maximum(g1 + fc1b_ref[...], 0.0)
    g2 = jnp.sum(g1 * fc2w_ref[...], axis=1, keepdims=True) + fc2b_ref[...]
    out_ref[...] = jax.nn.sigmoid(g2)


def _tc_final(s1, s2, mx, mn, bn3wr, bn3br, fc1w, fc1br, fc2wr, fc2br):
    return pl.pallas_call(
        _tc_final_body,
        out_shape=jax.ShapeDtypeStruct((1, 1), jnp.float32),
    )(s1, s2, mx, mn, bn3wr, bn3br, fc1w, fc1br, fc2wr, fc2br)


def kernel(x, edge_index, bn1_w, bn1_b, conv1_W, conv1_b, bn2_w, bn2_b,
           conv2_W, conv2_b, bn3_w, bn3_b, fc1_W, fc1_b, fc2_W, fc2_b):
    f32 = jnp.float32
    i32 = jnp.int32
    loop = jnp.arange(N, dtype=i32)
    dump = jnp.full((8,), ZROW, i32)
    er = jnp.concatenate([edge_index[0].astype(i32), loop, dump])
    ec = jnp.concatenate([edge_index[1].astype(i32), loop, dump])
    keep3 = jnp.asarray(KEEP_PAD).reshape(32, NBLK_A, 128)
    ewf = (jnp.arange(EPAD, dtype=i32) < E1).astype(f32)
    zflat = jnp.zeros((NT,), f32)
    zrows = jnp.zeros((128, 128), f32)

    degp, rowp4, colp = _sc_prep(er, ec, keep3, ewf, zflat)
    rowp4r = rowp4.reshape(4, 16, 2, NHALF, 128)
    col3 = colp.reshape(16, 2, NHALF, 128)

    xp = jnp.pad(x, ((0, NT - N), (0, 128 - x.shape[1])))
    w1p = jnp.pad(conv1_W, ((0, 128 - conv1_W.shape[0]), (0, 0)))
    bn1wp = jnp.pad(bn1_w, (0, 128 - bn1_w.shape[0])).reshape(1, 128)
    bn1bp = jnp.pad(bn1_b, (0, 128 - bn1_b.shape[0])).reshape(1, 128)
    s1x, s2x, dinv2d = _tc_prep(xp, degp.reshape(2, NT // 128, 128))
    dinv_c = dinv2d.reshape(NT, 1)

    b1r = conv1_b.reshape(1, 512)
    b2r = conv2_b.reshape(1, 512)
    bn2wr = bn2_w.reshape(1, 512)
    bn2br = bn2_b.reshape(1, 512)
    bn3wr = bn3_w.reshape(1, 512)
    bn3br = bn3_b.reshape(1, 512)
    fc1br = fc1_b.reshape(1, 512)
    fc2wr = fc2_W.reshape(1, 512)
    fc2br = fc2_b.reshape(1, 1)

    xn16 = _tc_xn(xp, s1x, s2x, bn1wp, bn1bp, dinv_c)
    aggx = _sc_agg16(xn16, rowp4, colp, zrows)
    s1, s2 = _tc_stats1(aggx, dinv_c, b1r, w1p)
    h2 = _tc_mm2(aggx, dinv_c, b1r, w1p, s1, s2, bn2wr, bn2br, conv2_W)
    agg2 = _sc_agg(h2.reshape(4 * NT, 128), rowp4r, col3, zrows)
    s1b, s2b, mx, mn = _tc_stats2(agg2, dinv_c, b2r)
    return _tc_final(s1b, s2b, mx, mn, bn3wr, bn3br, fc1_W, fc1br,
                     fc2wr, fc2br)
